# Initial kernel scaffold; baseline (speedup 1.0000x reference)
import jax
import jax.numpy as jnp
from jax.experimental import pallas as pl


def kernel(f_atoms, f_bonds, edge_index, b2revb, mol_ids, W_i, W_h, W_o, b_o, W_f1, b_f1, W_f2, b_f2):
    raise NotImplementedError("write your pallas kernel here")


# speedup vs baseline: 1.1746x; 1.1746x over previous
import functools

import jax
import jax.numpy as jnp
from jax import lax
from jax.experimental import pallas as pl
from jax.experimental.pallas import tpu as pltpu
from jax.experimental.pallas import tpu_sc as plsc

NB = 320000
NA = 10000
NAP = 10240
AF = 128
DIN = 144
HID = 128
NM = 500
NT = 1

NC = 2
NS = 16
NBH = NB // NC
BT = NBH // NS
K = 80
NCH = BT // K
ASTRIPE = NAP // NS
CR = K

_HIGH = lax.Precision.HIGHEST


def _dot(a, b):
    return lax.dot_general(a, b, (((1,), (0,)), ((), ())),
                           preferred_element_type=jnp.float32,
                           precision=_HIGH)


_TC_R = 2000


def _tc1_body(fb_ref, wi_ref, wh_ref, inp_ref, mh_ref):
    inp = _dot(fb_ref[...], wi_ref[...])
    inp_ref[...] = inp
    mh_ref[...] = _dot(jnp.maximum(inp, 0.0), wh_ref[...])


def _tc1(f_bonds, W_i, W_h):
    n = NB // _TC_R
    full = jax.ShapeDtypeStruct((NB, HID), jnp.float32)
    return pl.pallas_call(
        _tc1_body,
        grid=(n,),
        in_specs=[
            pl.BlockSpec((_TC_R, DIN), lambda i: (i, 0)),
            pl.BlockSpec((DIN, HID), lambda i: (0, 0)),
            pl.BlockSpec((HID, HID), lambda i: (0, 0)),
        ],
        out_specs=[pl.BlockSpec((_TC_R, HID), lambda i: (i, 0))] * 2,
        out_shape=[full, full],
    )(f_bonds, W_i, W_h)


def _tc2_body(m_ref, wh_ref, mh_ref):
    mh_ref[...] = _dot(m_ref[...], wh_ref[...])


def _tc2(m, W_h):
    n = NB // _TC_R
    return pl.pallas_call(
        _tc2_body,
        grid=(n,),
        in_specs=[
            pl.BlockSpec((_TC_R, HID), lambda i: (i, 0)),
            pl.BlockSpec((HID, HID), lambda i: (0, 0)),
        ],
        out_specs=pl.BlockSpec((_TC_R, HID), lambda i: (i, 0)),
        out_shape=jax.ShapeDtypeStruct((NB, HID), jnp.float32),
    )(m, W_h)


_TC3_A = 1000


def _tc3_body(fa_ref, r_ref, mid_ref, wot_ref, wob_ref,
              bo_ref, wf1_ref, bf1_ref, wf2_ref, bf2_ref, out_ref,
              acc_ref, cnt_ref):
    i = pl.program_id(0)

    @pl.when(i == 0)
    def _():
        acc_ref[...] = jnp.zeros_like(acc_ref)
        cnt_ref[...] = jnp.zeros_like(cnt_ref)

    a_blk = r_ref[0] + r_ref[1]
    ah = _dot(fa_ref[...], wot_ref[...]) + _dot(a_blk, wob_ref[...])
    ah = jnp.maximum(ah + bo_ref[...], 0.0)

    ids = mid_ref[0, 0, :]
    onehot = (lax.broadcasted_iota(jnp.int32, (NM, _TC3_A), 0)
              == ids[None, :]).astype(jnp.float32)
    acc_ref[...] += _dot(onehot, ah)
    cnt_ref[...] += jnp.broadcast_to(
        jnp.sum(onehot, axis=1, keepdims=True), (NM, HID))

    mol_vecs = acc_ref[...] / jnp.maximum(cnt_ref[...], 1.0)
    h = jnp.maximum(_dot(mol_vecs, wf1_ref[...]) + bf1_ref[...], 0.0)
    out_ref[...] = _dot(h, wf2_ref[...]) + bf2_ref[...]


def _tc3(f_atoms, r, mids3d, W_o, b_o, W_f1, b_f1, W_f2, b_f2):
    n = NA // _TC3_A
    return pl.pallas_call(
        _tc3_body,
        grid=(n,),
        in_specs=[
            pl.BlockSpec((_TC3_A, AF), lambda i: (i, 0)),
            pl.BlockSpec((NC, _TC3_A, HID), lambda i: (0, i, 0)),
            pl.BlockSpec((1, 1, _TC3_A), lambda i: (i, 0, 0)),
            pl.BlockSpec((AF, HID), lambda i: (0, 0)),
            pl.BlockSpec((HID, HID), lambda i: (0, 0)),
            pl.BlockSpec((1, HID), lambda i: (0, 0)),
            pl.BlockSpec((HID, HID), lambda i: (0, 0)),
            pl.BlockSpec((1, HID), lambda i: (0, 0)),
            pl.BlockSpec((HID, NT), lambda i: (0, 0)),
            pl.BlockSpec((1, NT), lambda i: (0, 0)),
        ],
        out_specs=pl.BlockSpec((NM, NT), lambda i: (0, 0)),
        out_shape=jax.ShapeDtypeStruct((NM, NT), jnp.float32),
        scratch_shapes=[
            pltpu.VMEM((NM, HID), jnp.float32),
            pltpu.VMEM((NM, HID), jnp.float32),
        ],
    )(f_atoms, r, mids3d, W_o[:AF], W_o[AF:],
      b_o, W_f1, b_f1, W_f2, b_f2)


_SC_MESH = plsc.VectorSubcoreMesh(core_axis_name="c", subcore_axis_name="s")


def _sc_scatter(dst, mh):
    scratch = [
        pltpu.VMEM((K,), jnp.int32),
        pltpu.VMEM((K, HID), jnp.float32),
        pltpu.VMEM_SHARED((NAP, HID), jnp.float32),
    ]

    @functools.partial(
        pl.kernel,
        out_type=jax.ShapeDtypeStruct((NC, NAP, HID), jnp.float32),
        mesh=_SC_MESH, scratch_types=scratch)
    def scatter_k(dst_h, mh_h, out_h, dstv, rows, a_sh):
        cid = lax.axis_index("c")
        sid = lax.axis_index("s")
        base = cid * NBH + sid * BT

        z16 = jnp.zeros((16,), jnp.float32)

        @pl.loop(0, CR)
        def _(rr):
            for c0 in range(0, HID, 16):
                rows[rr, pl.ds(c0, 16)] = z16

        for j in range(ASTRIPE // CR):
            pltpu.sync_copy(rows, a_sh.at[pl.ds(sid * ASTRIPE + j * CR, CR)])
        plsc.subcore_barrier()

        @pl.loop(0, NCH)
        def _(ci):
            b = base + ci * K
            pltpu.sync_copy(dst_h.at[pl.ds(b, K)], dstv)
            pltpu.sync_copy(mh_h.at[pl.ds(b, K)], rows)
            pltpu.sync_copy(rows, a_sh.at[dstv], add=True)

        plsc.subcore_barrier()
        for j in range(ASTRIPE // CR):
            r0 = sid * ASTRIPE + j * CR
            pltpu.sync_copy(a_sh.at[pl.ds(r0, CR)], rows)
            pltpu.sync_copy(rows, out_h.at[cid, pl.ds(r0, CR)])

    return scatter_k(dst, mh)


def _sc_gather(p0f, p1f, src, rev, mh, inp):
    scratch = [
        pltpu.VMEM((K,), jnp.int32),
        pltpu.VMEM((K,), jnp.int32),
        pltpu.VMEM((K, HID), jnp.float32),
        pltpu.VMEM((K, HID), jnp.float32),
        pltpu.VMEM((K, HID), jnp.float32),
        pltpu.VMEM((K, HID), jnp.float32),
    ]

    @functools.partial(
        pl.kernel,
        out_type=jax.ShapeDtypeStruct((NB, HID), jnp.float32),
        mesh=_SC_MESH, scratch_types=scratch)
    def gather_k(p0_h, p1_h, src_h, rev_h, mh_h, inp_h, out_h,
                 srcv, revv, gv, pv, rv, ov):
        cid = lax.axis_index("c")
        sid = lax.axis_index("s")
        base = cid * NBH + sid * BT

        @pl.loop(0, NCH)
        def _(ci):
            b = base + ci * K
            pltpu.sync_copy(src_h.at[pl.ds(b, K)], srcv)
            pltpu.sync_copy(rev_h.at[pl.ds(b, K)], revv)
            pltpu.sync_copy(p0_h.at[srcv], gv)
            pltpu.sync_copy(p1_h.at[srcv], pv)
            pltpu.sync_copy(mh_h.at[revv], rv)
            pltpu.sync_copy(inp_h.at[pl.ds(b, K)], ov)

            @pl.loop(0, K)
            def _(rr):
                for c0 in range(0, HID, 16):
                    s = (rr, pl.ds(c0, 16))
                    ov[s] = jnp.maximum(ov[s] + gv[s] + pv[s] - rv[s], 0.0)

            pltpu.sync_copy(ov, out_h.at[pl.ds(b, K)])

    return gather_k(p0f, p1f, src, rev, mh, inp)


def kernel(f_atoms, f_bonds, edge_index, b2revb, mol_ids,
           W_i, W_h, W_o, b_o, W_f1, b_f1, W_f2, b_f2):
    ei = edge_index.astype(jnp.int32)
    src = ei[0]
    dst = ei[1]
    rev = b2revb.astype(jnp.int32)
    mids3d = mol_ids.astype(jnp.int32).reshape(NA // _TC3_A, 1, _TC3_A)

    inp, mh0 = _tc1(f_bonds, W_i, W_h)
    p0 = _sc_scatter(dst, mh0)
    m1 = _sc_gather(p0[0], p0[1], src, rev, mh0, inp)
    mh1 = _tc2(m1, W_h)
    p1 = _sc_scatter(dst, mh1)
    m2 = _sc_gather(p1[0], p1[1], src, rev, mh1, inp)
    r = _sc_scatter(dst, m2)
    out = _tc3(f_atoms, r, mids3d,
               W_o, b_o.reshape(1, HID), W_f1, b_f1.reshape(1, HID),
               W_f2, b_f2.reshape(1, NT))
    return out


# speedup vs baseline: 1.7880x; 1.5222x over previous
import functools

import jax
import jax.numpy as jnp
from jax import lax
from jax.experimental import pallas as pl
from jax.experimental.pallas import tpu as pltpu
from jax.experimental.pallas import tpu_sc as plsc

NB = 320000
NA = 10000
NAP = 10240
AF = 128
DIN = 144
HID = 128
NM = 500
NT = 1

NC = 2
NS = 16
NBH = NB // NC
BT = NBH // NS
K = 80
NCH = BT // K
ASTRIPE = NAP // NS
CR = K

_HIGH = lax.Precision.HIGHEST


def _dot(a, b, precision=_HIGH):
    return lax.dot_general(a, b, (((1,), (0,)), ((), ())),
                           preferred_element_type=jnp.float32,
                           precision=precision)


def _split_bf16(x):
    hi = x.astype(jnp.bfloat16)
    lo = (x - hi.astype(jnp.float32)).astype(jnp.bfloat16)
    return hi, lo


def _dot3(a, b):
    ah, al = _split_bf16(a)
    bh, bl = _split_bf16(b)
    d = lambda x, y: _dot(x, y, precision=None)
    return d(ah, bh) + d(ah, bl) + d(al, bh)


_TC_R = 2000


def _tc1_body(fb_ref, wi_ref, wh_ref, inp_ref, mh_ref):
    inp = _dot3(fb_ref[...], wi_ref[...])
    inp_ref[...] = inp
    mh_ref[...] = _dot3(jnp.maximum(inp, 0.0), wh_ref[...])


def _tc1(f_bonds, W_i, W_h):
    n = NB // _TC_R
    full = jax.ShapeDtypeStruct((NB, HID), jnp.float32)
    return pl.pallas_call(
        _tc1_body,
        grid=(n,),
        in_specs=[
            pl.BlockSpec((_TC_R, DIN), lambda i: (i, 0)),
            pl.BlockSpec((DIN, HID), lambda i: (0, 0)),
            pl.BlockSpec((HID, HID), lambda i: (0, 0)),
        ],
        out_specs=[pl.BlockSpec((_TC_R, HID), lambda i: (i, 0))] * 2,
        out_shape=[full, full],
    )(f_bonds, W_i, W_h)


def _tc2_body(m_ref, wh_ref, mh_ref):
    mh_ref[...] = _dot3(m_ref[...], wh_ref[...])


def _tc2(m, W_h):
    n = NB // _TC_R
    return pl.pallas_call(
        _tc2_body,
        grid=(n,),
        in_specs=[
            pl.BlockSpec((_TC_R, HID), lambda i: (i, 0)),
            pl.BlockSpec((HID, HID), lambda i: (0, 0)),
        ],
        out_specs=pl.BlockSpec((_TC_R, HID), lambda i: (i, 0)),
        out_shape=jax.ShapeDtypeStruct((NB, HID), jnp.float32),
    )(m, W_h)


_TC3_A = 1000


def _tc3_body(fa_ref, r_ref, mid_ref, wot_ref, wob_ref,
              bo_ref, wf1_ref, bf1_ref, wf2_ref, bf2_ref, out_ref,
              acc_ref, cnt_ref):
    i = pl.program_id(0)

    @pl.when(i == 0)
    def _():
        acc_ref[...] = jnp.zeros_like(acc_ref)
        cnt_ref[...] = jnp.zeros_like(cnt_ref)

    a_blk = r_ref[0] + r_ref[1]
    ah = _dot(fa_ref[...], wot_ref[...]) + _dot(a_blk, wob_ref[...])
    ah = jnp.maximum(ah + bo_ref[...], 0.0)

    ids = mid_ref[0, 0, :]
    onehot = (lax.broadcasted_iota(jnp.int32, (NM, _TC3_A), 0)
              == ids[None, :]).astype(jnp.float32)
    oh_bf = onehot.astype(jnp.bfloat16)
    ah_hi, ah_lo = _split_bf16(ah)
    acc_ref[...] += (_dot(oh_bf, ah_hi, precision=None)
                     + _dot(oh_bf, ah_lo, precision=None))
    cnt_ref[...] += jnp.broadcast_to(
        jnp.sum(onehot, axis=1, keepdims=True), (NM, HID))

    mol_vecs = acc_ref[...] / jnp.maximum(cnt_ref[...], 1.0)
    h = jnp.maximum(_dot(mol_vecs, wf1_ref[...]) + bf1_ref[...], 0.0)
    out_ref[...] = _dot(h, wf2_ref[...]) + bf2_ref[...]


def _tc3(f_atoms, r, mids3d, W_o, b_o, W_f1, b_f1, W_f2, b_f2):
    n = NA // _TC3_A
    return pl.pallas_call(
        _tc3_body,
        grid=(n,),
        in_specs=[
            pl.BlockSpec((_TC3_A, AF), lambda i: (i, 0)),
            pl.BlockSpec((NC, _TC3_A, HID), lambda i: (0, i, 0)),
            pl.BlockSpec((1, 1, _TC3_A), lambda i: (i, 0, 0)),
            pl.BlockSpec((AF, HID), lambda i: (0, 0)),
            pl.BlockSpec((HID, HID), lambda i: (0, 0)),
            pl.BlockSpec((1, HID), lambda i: (0, 0)),
            pl.BlockSpec((HID, HID), lambda i: (0, 0)),
            pl.BlockSpec((1, HID), lambda i: (0, 0)),
            pl.BlockSpec((HID, NT), lambda i: (0, 0)),
            pl.BlockSpec((1, NT), lambda i: (0, 0)),
        ],
        out_specs=pl.BlockSpec((NM, NT), lambda i: (0, 0)),
        out_shape=jax.ShapeDtypeStruct((NM, NT), jnp.float32),
        scratch_shapes=[
            pltpu.VMEM((NM, HID), jnp.float32),
            pltpu.VMEM((NM, HID), jnp.float32),
        ],
    )(f_atoms, r, mids3d, W_o[:AF], W_o[AF:],
      b_o, W_f1, b_f1, W_f2, b_f2)


_SC_MESH = plsc.VectorSubcoreMesh(core_axis_name="c", subcore_axis_name="s")


def _sc_scatter(dst, mh):
    scratch = [
        pltpu.VMEM((K,), jnp.int32),
        pltpu.VMEM((K,), jnp.int32),
        pltpu.VMEM((K, HID), jnp.float32),
        pltpu.VMEM((K, HID), jnp.float32),
        pltpu.VMEM_SHARED((NAP, HID), jnp.float32),
        pltpu.SemaphoreType.DMA((4,)),
    ]

    @functools.partial(
        pl.kernel,
        out_type=jax.ShapeDtypeStruct((NC, NAP, HID), jnp.float32),
        mesh=_SC_MESH, scratch_types=scratch)
    def scatter_k(dst_h, mh_h, out_h, dstv0, dstv1, rows0, rows1, a_sh,
                  sems):
        cid = lax.axis_index("c")
        sid = lax.axis_index("s")
        base = cid * NBH + sid * BT

        z16 = jnp.zeros((16,), jnp.float32)

        @pl.loop(0, CR)
        def _(rr):
            for c0 in range(0, HID, 16):
                rows0[rr, pl.ds(c0, 16)] = z16

        zd = [pltpu.async_copy(
                  rows0, a_sh.at[pl.ds(sid * ASTRIPE + j * CR, CR)],
                  sems.at[0])
              for j in range(ASTRIPE // CR)]
        for d in zd:
            d.wait()
        plsc.subcore_barrier()

        def loads(ci, dv, rw, sem):
            b = base + ci * K
            return (pltpu.async_copy(dst_h.at[pl.ds(b, K)], dv, sem),
                    pltpu.async_copy(mh_h.at[pl.ds(b, K)], rw, sem))

        @pl.loop(0, NCH - 1, step=2)
        def _(ci):
            l0 = loads(ci, dstv0, rows0, sems.at[0])
            l1 = loads(ci + 1, dstv1, rows1, sems.at[1])
            for d in l0:
                d.wait()
            s0 = pltpu.async_copy(rows0, a_sh.at[dstv0], sems.at[2],
                                  add=True)
            for d in l1:
                d.wait()
            s1 = pltpu.async_copy(rows1, a_sh.at[dstv1], sems.at[3],
                                  add=True)
            s0.wait()
            s1.wait()

        for d in loads(NCH - 1, dstv0, rows0, sems.at[0]):
            d.wait()
        pltpu.sync_copy(rows0, a_sh.at[dstv0], add=True)

        plsc.subcore_barrier()
        for j in range(0, ASTRIPE // CR, 2):
            r0 = sid * ASTRIPE + j * CR
            r1 = r0 + CR
            c0 = pltpu.async_copy(a_sh.at[pl.ds(r0, CR)], rows0, sems.at[0])
            c1 = pltpu.async_copy(a_sh.at[pl.ds(r1, CR)], rows1, sems.at[1])
            c0.wait()
            o0 = pltpu.async_copy(rows0, out_h.at[cid, pl.ds(r0, CR)],
                                  sems.at[2])
            c1.wait()
            o1 = pltpu.async_copy(rows1, out_h.at[cid, pl.ds(r1, CR)],
                                  sems.at[3])
            o0.wait()
            o1.wait()

    return scatter_k(dst, mh)


def _sc_gather(p0f, p1f, src, rev, mh, inp):
    scratch = [
        pltpu.VMEM((K,), jnp.int32),
        pltpu.VMEM((K,), jnp.int32),
        pltpu.VMEM((K,), jnp.int32),
        pltpu.VMEM((K,), jnp.int32),
        pltpu.VMEM((K, HID), jnp.float32),
        pltpu.VMEM((K, HID), jnp.float32),
        pltpu.VMEM((K, HID), jnp.float32),
        pltpu.VMEM((K, HID), jnp.float32),
        pltpu.VMEM((K, HID), jnp.float32),
        pltpu.VMEM((K, HID), jnp.float32),
        pltpu.VMEM((K, HID), jnp.float32),
        pltpu.VMEM((K, HID), jnp.float32),
        pltpu.SemaphoreType.DMA((6,)),
    ]

    @functools.partial(
        pl.kernel,
        out_type=jax.ShapeDtypeStruct((NB, HID), jnp.float32),
        mesh=_SC_MESH, scratch_types=scratch)
    def gather_k(p0_h, p1_h, src_h, rev_h, mh_h, inp_h, out_h,
                 srcv0, srcv1, revv0, revv1, gv0, gv1, pv0, pv1,
                 rv0, rv1, ov0, ov1, sems):
        cid = lax.axis_index("c")
        sid = lax.axis_index("s")
        base = cid * NBH + sid * BT

        def idx_loads(ci, sv, rvv, sem):
            b = base + ci * K
            return (pltpu.async_copy(src_h.at[pl.ds(b, K)], sv, sem),
                    pltpu.async_copy(rev_h.at[pl.ds(b, K)], rvv, sem))

        def dat_loads(ci, sv, rvv, gv, pv, rv, ov, sem):
            b = base + ci * K
            return (pltpu.async_copy(p0_h.at[sv], gv, sem),
                    pltpu.async_copy(p1_h.at[sv], pv, sem),
                    pltpu.async_copy(mh_h.at[rvv], rv, sem),
                    pltpu.async_copy(inp_h.at[pl.ds(b, K)], ov, sem))

        def compute(gv, pv, rv, ov):
            @pl.loop(0, K)
            def _(rr):
                for c0 in range(0, HID, 16):
                    s = (rr, pl.ds(c0, 16))
                    ov[s] = jnp.maximum(ov[s] + gv[s] + pv[s] - rv[s], 0.0)

        @pl.loop(0, NCH - 1, step=2)
        def _(ci):
            i0 = idx_loads(ci, srcv0, revv0, sems.at[0])
            i1 = idx_loads(ci + 1, srcv1, revv1, sems.at[1])
            for d in i0:
                d.wait()
            d0 = dat_loads(ci, srcv0, revv0, gv0, pv0, rv0, ov0, sems.at[2])
            for d in i1:
                d.wait()
            d1 = dat_loads(ci + 1, srcv1, revv1, gv1, pv1, rv1, ov1,
                           sems.at[3])
            for d in d0:
                d.wait()
            compute(gv0, pv0, rv0, ov0)
            o0 = pltpu.async_copy(ov0, out_h.at[pl.ds(base + ci * K, K)],
                                  sems.at[4])
            for d in d1:
                d.wait()
            compute(gv1, pv1, rv1, ov1)
            o1 = pltpu.async_copy(ov1,
                                  out_h.at[pl.ds(base + (ci + 1) * K, K)],
                                  sems.at[5])
            o0.wait()
            o1.wait()

        ce = NCH - 1
        for d in idx_loads(ce, srcv0, revv0, sems.at[0]):
            d.wait()
        for d in dat_loads(ce, srcv0, revv0, gv0, pv0, rv0, ov0, sems.at[2]):
            d.wait()
        compute(gv0, pv0, rv0, ov0)
        pltpu.sync_copy(ov0, out_h.at[pl.ds(base + ce * K, K)])

    return gather_k(p0f, p1f, src, rev, mh, inp)


def kernel(f_atoms, f_bonds, edge_index, b2revb, mol_ids,
           W_i, W_h, W_o, b_o, W_f1, b_f1, W_f2, b_f2):
    ei = edge_index.astype(jnp.int32)
    src = ei[0]
    dst = ei[1]
    rev = b2revb.astype(jnp.int32)
    mids3d = mol_ids.astype(jnp.int32).reshape(NA // _TC3_A, 1, _TC3_A)

    inp, mh0 = _tc1(f_bonds, W_i, W_h)
    p0 = _sc_scatter(dst, mh0)
    m1 = _sc_gather(p0[0], p0[1], src, rev, mh0, inp)
    mh1 = _tc2(m1, W_h)
    p1 = _sc_scatter(dst, mh1)
    m2 = _sc_gather(p1[0], p1[1], src, rev, mh1, inp)
    r = _sc_scatter(dst, m2)
    out = _tc3(f_atoms, r, mids3d,
               W_o, b_o.reshape(1, HID), W_f1, b_f1.reshape(1, HID),
               W_f2, b_f2.reshape(1, NT))
    return out


# speedup vs baseline: 2.0283x; 1.1344x over previous
import functools

import jax
import jax.numpy as jnp
from jax import lax
from jax.experimental import pallas as pl
from jax.experimental.pallas import tpu as pltpu
from jax.experimental.pallas import tpu_sc as plsc

NB = 320000
NA = 10000
NAP = 10240
AF = 128
DIN = 144
HID = 128
NM = 500
NT = 1

NC = 2
NS = 16
NBH = NB // NC
BT = NBH // NS
K = 80
NCH = BT // K
ASTRIPE = NAP // NS
CR = K

_HIGH = lax.Precision.HIGHEST


def _dot(a, b, precision=_HIGH):
    return lax.dot_general(a, b, (((1,), (0,)), ((), ())),
                           preferred_element_type=jnp.float32,
                           precision=precision)


def _split_bf16(x):
    hi = x.astype(jnp.bfloat16)
    lo = (x - hi.astype(jnp.float32)).astype(jnp.bfloat16)
    return hi, lo


def _dot3(a, b):
    ah, al = _split_bf16(a)
    bh, bl = _split_bf16(b)
    d = lambda x, y: _dot(x, y, precision=None)
    return d(ah, bh) + d(ah, bl) + d(al, bh)


_TC_R = 2000


def _tc1_body(fb_ref, wi_ref, wh_ref, inp_ref, mh_ref):
    inp = _dot3(fb_ref[...], wi_ref[...])
    inp_ref[...] = inp
    mh_ref[...] = _dot3(jnp.maximum(inp, 0.0), wh_ref[...])


def _tc1(f_bonds, W_i, W_h):
    n = NB // _TC_R
    full = jax.ShapeDtypeStruct((NB, HID), jnp.float32)
    return pl.pallas_call(
        _tc1_body,
        grid=(n,),
        in_specs=[
            pl.BlockSpec((_TC_R, DIN), lambda i: (i, 0)),
            pl.BlockSpec((DIN, HID), lambda i: (0, 0)),
            pl.BlockSpec((HID, HID), lambda i: (0, 0)),
        ],
        out_specs=[pl.BlockSpec((_TC_R, HID), lambda i: (i, 0))] * 2,
        out_shape=[full, full],
    )(f_bonds, W_i, W_h)


def _tc2_body(m_ref, wh_ref, mh_ref):
    mh_ref[...] = _dot3(m_ref[...], wh_ref[...])


def _tc2(m, W_h):
    n = NB // _TC_R
    return pl.pallas_call(
        _tc2_body,
        grid=(n,),
        in_specs=[
            pl.BlockSpec((_TC_R, HID), lambda i: (i, 0)),
            pl.BlockSpec((HID, HID), lambda i: (0, 0)),
        ],
        out_specs=pl.BlockSpec((_TC_R, HID), lambda i: (i, 0)),
        out_shape=jax.ShapeDtypeStruct((NB, HID), jnp.float32),
    )(m, W_h)


_TC3_A = 1000


def _tc3_body(fa_ref, r_ref, mid_ref, wot_ref, wob_ref,
              bo_ref, wf1_ref, bf1_ref, wf2_ref, bf2_ref, out_ref,
              acc_ref, cnt_ref):
    i = pl.program_id(0)

    @pl.when(i == 0)
    def _():
        acc_ref[...] = jnp.zeros_like(acc_ref)
        cnt_ref[...] = jnp.zeros_like(cnt_ref)

    a_blk = r_ref[0] + r_ref[1]
    ah = _dot(fa_ref[...], wot_ref[...]) + _dot(a_blk, wob_ref[...])
    ah = jnp.maximum(ah + bo_ref[...], 0.0)

    ids = mid_ref[0, 0, :]
    onehot = (lax.broadcasted_iota(jnp.int32, (NM, _TC3_A), 0)
              == ids[None, :]).astype(jnp.float32)
    oh_bf = onehot.astype(jnp.bfloat16)
    ah_hi, ah_lo = _split_bf16(ah)
    acc_ref[...] += (_dot(oh_bf, ah_hi, precision=None)
                     + _dot(oh_bf, ah_lo, precision=None))
    cnt_ref[...] += jnp.broadcast_to(
        jnp.sum(onehot, axis=1, keepdims=True), (NM, HID))

    mol_vecs = acc_ref[...] / jnp.maximum(cnt_ref[...], 1.0)
    h = jnp.maximum(_dot(mol_vecs, wf1_ref[...]) + bf1_ref[...], 0.0)
    out_ref[...] = _dot(h, wf2_ref[...]) + bf2_ref[...]


def _tc3(f_atoms, r, mids3d, W_o, b_o, W_f1, b_f1, W_f2, b_f2):
    n = NA // _TC3_A
    return pl.pallas_call(
        _tc3_body,
        grid=(n,),
        in_specs=[
            pl.BlockSpec((_TC3_A, AF), lambda i: (i, 0)),
            pl.BlockSpec((NC, _TC3_A, HID), lambda i: (0, i, 0)),
            pl.BlockSpec((1, 1, _TC3_A), lambda i: (i, 0, 0)),
            pl.BlockSpec((AF, HID), lambda i: (0, 0)),
            pl.BlockSpec((HID, HID), lambda i: (0, 0)),
            pl.BlockSpec((1, HID), lambda i: (0, 0)),
            pl.BlockSpec((HID, HID), lambda i: (0, 0)),
            pl.BlockSpec((1, HID), lambda i: (0, 0)),
            pl.BlockSpec((HID, NT), lambda i: (0, 0)),
            pl.BlockSpec((1, NT), lambda i: (0, 0)),
        ],
        out_specs=pl.BlockSpec((NM, NT), lambda i: (0, 0)),
        out_shape=jax.ShapeDtypeStruct((NM, NT), jnp.float32),
        scratch_shapes=[
            pltpu.VMEM((NM, HID), jnp.float32),
            pltpu.VMEM((NM, HID), jnp.float32),
        ],
    )(f_atoms, r, mids3d, W_o[:AF], W_o[AF:],
      b_o, W_f1, b_f1, W_f2, b_f2)


def _tcm_body(p_ref, a_ref):
    a_ref[...] = p_ref[0] + p_ref[1]


def _tc_merge(p):
    n = 8
    blk = NAP // n
    return pl.pallas_call(
        _tcm_body,
        grid=(n,),
        in_specs=[pl.BlockSpec((NC, blk, HID), lambda i: (0, i, 0))],
        out_specs=pl.BlockSpec((blk, HID), lambda i: (i, 0)),
        out_shape=jax.ShapeDtypeStruct((NAP, HID), jnp.float32),
    )(p)


_SC_MESH = plsc.VectorSubcoreMesh(core_axis_name="c", subcore_axis_name="s")


def _sc_scatter(dst, mh):
    scratch = [
        pltpu.VMEM((K,), jnp.int32),
        pltpu.VMEM((K,), jnp.int32),
        pltpu.VMEM((K, HID), jnp.float32),
        pltpu.VMEM((K, HID), jnp.float32),
        pltpu.VMEM_SHARED((NAP, HID), jnp.float32),
        pltpu.SemaphoreType.DMA((4,)),
    ]

    @functools.partial(
        pl.kernel,
        out_type=jax.ShapeDtypeStruct((NC, NAP, HID), jnp.float32),
        mesh=_SC_MESH, scratch_types=scratch)
    def scatter_k(dst_h, mh_h, out_h, dstv0, dstv1, rows0, rows1, a_sh,
                  sems):
        cid = lax.axis_index("c")
        sid = lax.axis_index("s")
        base = cid * NBH + sid * BT

        z16 = jnp.zeros((16,), jnp.float32)

        @pl.loop(0, CR)
        def _(rr):
            for c0 in range(0, HID, 16):
                rows0[rr, pl.ds(c0, 16)] = z16

        zd = [pltpu.async_copy(
                  rows0, a_sh.at[pl.ds(sid * ASTRIPE + j * CR, CR)],
                  sems.at[0])
              for j in range(ASTRIPE // CR)]
        for d in zd:
            d.wait()
        plsc.subcore_barrier()

        def loads(ci, dv, rw, sem):
            b = base + ci * K
            return (pltpu.async_copy(dst_h.at[pl.ds(b, K)], dv, sem),
                    pltpu.async_copy(mh_h.at[pl.ds(b, K)], rw, sem))

        @pl.loop(0, NCH - 1, step=2)
        def _(ci):
            l0 = loads(ci, dstv0, rows0, sems.at[0])
            l1 = loads(ci + 1, dstv1, rows1, sems.at[1])
            for d in l0:
                d.wait()
            s0 = pltpu.async_copy(rows0, a_sh.at[dstv0], sems.at[2],
                                  add=True)
            for d in l1:
                d.wait()
            s1 = pltpu.async_copy(rows1, a_sh.at[dstv1], sems.at[3],
                                  add=True)
            s0.wait()
            s1.wait()

        for d in loads(NCH - 1, dstv0, rows0, sems.at[0]):
            d.wait()
        pltpu.sync_copy(rows0, a_sh.at[dstv0], add=True)

        plsc.subcore_barrier()
        for j in range(0, ASTRIPE // CR, 2):
            r0 = sid * ASTRIPE + j * CR
            r1 = r0 + CR
            c0 = pltpu.async_copy(a_sh.at[pl.ds(r0, CR)], rows0, sems.at[0])
            c1 = pltpu.async_copy(a_sh.at[pl.ds(r1, CR)], rows1, sems.at[1])
            c0.wait()
            o0 = pltpu.async_copy(rows0, out_h.at[cid, pl.ds(r0, CR)],
                                  sems.at[2])
            c1.wait()
            o1 = pltpu.async_copy(rows1, out_h.at[cid, pl.ds(r1, CR)],
                                  sems.at[3])
            o0.wait()
            o1.wait()

    return scatter_k(dst, mh)


def _sc_gather(af, src, rev, mh, inp):
    scratch = [
        pltpu.VMEM((K,), jnp.int32),
        pltpu.VMEM((K,), jnp.int32),
        pltpu.VMEM((K,), jnp.int32),
        pltpu.VMEM((K,), jnp.int32),
        pltpu.VMEM((K, HID), jnp.float32),
        pltpu.VMEM((K, HID), jnp.float32),
        pltpu.VMEM((K, HID), jnp.float32),
        pltpu.VMEM((K, HID), jnp.float32),
        pltpu.VMEM((K, HID), jnp.float32),
        pltpu.VMEM((K, HID), jnp.float32),
        pltpu.SemaphoreType.DMA((6,)),
    ]

    @functools.partial(
        pl.kernel,
        out_type=jax.ShapeDtypeStruct((NB, HID), jnp.float32),
        mesh=_SC_MESH, scratch_types=scratch)
    def gather_k(af_h, src_h, rev_h, mh_h, inp_h, out_h,
                 srcv0, srcv1, revv0, revv1, gv0, gv1,
                 rv0, rv1, ov0, ov1, sems):
        cid = lax.axis_index("c")
        sid = lax.axis_index("s")
        base = cid * NBH + sid * BT

        def idx_loads(ci, sv, rvv, sem):
            b = base + ci * K
            return (pltpu.async_copy(src_h.at[pl.ds(b, K)], sv, sem),
                    pltpu.async_copy(rev_h.at[pl.ds(b, K)], rvv, sem))

        def dat_loads(ci, sv, rvv, gv, rv, ov, sem):
            b = base + ci * K
            return (pltpu.async_copy(af_h.at[sv], gv, sem),
                    pltpu.async_copy(mh_h.at[rvv], rv, sem),
                    pltpu.async_copy(inp_h.at[pl.ds(b, K)], ov, sem))

        def compute(gv, rv, ov):
            @pl.loop(0, K)
            def _(rr):
                for c0 in range(0, HID, 16):
                    s = (rr, pl.ds(c0, 16))
                    ov[s] = jnp.maximum(ov[s] + gv[s] - rv[s], 0.0)

        @pl.loop(0, NCH - 1, step=2)
        def _(ci):
            i0 = idx_loads(ci, srcv0, revv0, sems.at[0])
            i1 = idx_loads(ci + 1, srcv1, revv1, sems.at[1])
            for d in i0:
                d.wait()
            d0 = dat_loads(ci, srcv0, revv0, gv0, rv0, ov0, sems.at[2])
            for d in i1:
                d.wait()
            d1 = dat_loads(ci + 1, srcv1, revv1, gv1, rv1, ov1,
                           sems.at[3])
            for d in d0:
                d.wait()
            compute(gv0, rv0, ov0)
            o0 = pltpu.async_copy(ov0, out_h.at[pl.ds(base + ci * K, K)],
                                  sems.at[4])
            for d in d1:
                d.wait()
            compute(gv1, rv1, ov1)
            o1 = pltpu.async_copy(ov1,
                                  out_h.at[pl.ds(base + (ci + 1) * K, K)],
                                  sems.at[5])
            o0.wait()
            o1.wait()

        ce = NCH - 1
        for d in idx_loads(ce, srcv0, revv0, sems.at[0]):
            d.wait()
        for d in dat_loads(ce, srcv0, revv0, gv0, rv0, ov0, sems.at[2]):
            d.wait()
        compute(gv0, rv0, ov0)
        pltpu.sync_copy(ov0, out_h.at[pl.ds(base + ce * K, K)])

    return gather_k(af, src, rev, mh, inp)


_KF = 40
_NCHF = BT // _KF
_CRF = 40


def _sc_gather_final(af, src, rev, dstf, mh, inp):
    KF, NCHF, CRF = _KF, _NCHF, _CRF
    scratch = [
        pltpu.VMEM((KF,), jnp.int32),
        pltpu.VMEM((KF,), jnp.int32),
        pltpu.VMEM((KF,), jnp.int32),
        pltpu.VMEM((KF,), jnp.int32),
        pltpu.VMEM((KF,), jnp.int32),
        pltpu.VMEM((KF,), jnp.int32),
        pltpu.VMEM((KF, HID), jnp.float32),
        pltpu.VMEM((KF, HID), jnp.float32),
        pltpu.VMEM((KF, HID), jnp.float32),
        pltpu.VMEM((KF, HID), jnp.float32),
        pltpu.VMEM((KF, HID), jnp.float32),
        pltpu.VMEM((KF, HID), jnp.float32),
        pltpu.VMEM_SHARED((NAP, HID), jnp.float32),
        pltpu.SemaphoreType.DMA((6,)),
    ]

    @functools.partial(
        pl.kernel,
        out_type=jax.ShapeDtypeStruct((NC, NAP, HID), jnp.float32),
        mesh=_SC_MESH, scratch_types=scratch)
    def gather_final_k(af_h, src_h, rev_h, dst_h, mh_h, inp_h, out_h,
                       srcv0, srcv1, revv0, revv1, dstv0, dstv1,
                       gv0, gv1, rv0, rv1, ov0, ov1, a2_sh, sems):
        cid = lax.axis_index("c")
        sid = lax.axis_index("s")
        base = cid * NBH + sid * BT

        z16 = jnp.zeros((16,), jnp.float32)

        @pl.loop(0, CRF)
        def _(rr):
            for c0 in range(0, HID, 16):
                ov0[rr, pl.ds(c0, 16)] = z16

        zd = [pltpu.async_copy(
                  ov0, a2_sh.at[pl.ds(sid * ASTRIPE + j * CRF, CRF)],
                  sems.at[0])
              for j in range(ASTRIPE // CRF)]
        for d in zd:
            d.wait()
        plsc.subcore_barrier()

        def idx_loads(ci, sv, rvv, dv, sem):
            b = base + ci * KF
            return (pltpu.async_copy(src_h.at[pl.ds(b, KF)], sv, sem),
                    pltpu.async_copy(rev_h.at[pl.ds(b, KF)], rvv, sem),
                    pltpu.async_copy(dst_h.at[pl.ds(b, KF)], dv, sem))

        def dat_loads(ci, sv, rvv, gv, rv, ov, sem):
            b = base + ci * KF
            return (pltpu.async_copy(af_h.at[sv], gv, sem),
                    pltpu.async_copy(mh_h.at[rvv], rv, sem),
                    pltpu.async_copy(inp_h.at[pl.ds(b, KF)], ov, sem))

        def compute(gv, rv, ov):
            @pl.loop(0, KF)
            def _(rr):
                for c0 in range(0, HID, 16):
                    s = (rr, pl.ds(c0, 16))
                    ov[s] = jnp.maximum(ov[s] + gv[s] - rv[s], 0.0)

        @pl.loop(0, NCHF, step=2)
        def _(ci):
            i0 = idx_loads(ci, srcv0, revv0, dstv0, sems.at[0])
            i1 = idx_loads(ci + 1, srcv1, revv1, dstv1, sems.at[1])
            for d in i0:
                d.wait()
            d0 = dat_loads(ci, srcv0, revv0, gv0, rv0, ov0, sems.at[2])
            for d in i1:
                d.wait()
            d1 = dat_loads(ci + 1, srcv1, revv1, gv1, rv1, ov1,
                           sems.at[3])
            for d in d0:
                d.wait()
            compute(gv0, rv0, ov0)
            s0 = pltpu.async_copy(ov0, a2_sh.at[dstv0], sems.at[4],
                                  add=True)
            for d in d1:
                d.wait()
            compute(gv1, rv1, ov1)
            s1 = pltpu.async_copy(ov1, a2_sh.at[dstv1], sems.at[5],
                                  add=True)
            s0.wait()
            s1.wait()

        plsc.subcore_barrier()
        for j in range(0, ASTRIPE // CRF, 2):
            r0 = sid * ASTRIPE + j * CRF
            r1 = r0 + CRF
            c0 = pltpu.async_copy(a2_sh.at[pl.ds(r0, CRF)], ov0, sems.at[0])
            c1 = pltpu.async_copy(a2_sh.at[pl.ds(r1, CRF)], ov1, sems.at[1])
            c0.wait()
            o0 = pltpu.async_copy(ov0, out_h.at[cid, pl.ds(r0, CRF)],
                                  sems.at[2])
            c1.wait()
            o1 = pltpu.async_copy(ov1, out_h.at[cid, pl.ds(r1, CRF)],
                                  sems.at[3])
            o0.wait()
            o1.wait()

    return gather_final_k(af, src, rev, dstf, mh, inp)


def kernel(f_atoms, f_bonds, edge_index, b2revb, mol_ids,
           W_i, W_h, W_o, b_o, W_f1, b_f1, W_f2, b_f2):
    ei = edge_index.astype(jnp.int32)
    src = ei[0]
    dst = ei[1]
    rev = b2revb.astype(jnp.int32)
    mids3d = mol_ids.astype(jnp.int32).reshape(NA // _TC3_A, 1, _TC3_A)

    inp, mh0 = _tc1(f_bonds, W_i, W_h)
    p0 = _sc_scatter(dst, mh0)
    m1 = _sc_gather(_tc_merge(p0), src, rev, mh0, inp)
    mh1 = _tc2(m1, W_h)
    p1 = _sc_scatter(dst, mh1)
    r = _sc_gather_final(_tc_merge(p1), src, rev, dst, mh1, inp)
    out = _tc3(f_atoms, r, mids3d,
               W_o, b_o.reshape(1, HID), W_f1, b_f1.reshape(1, HID),
               W_f2, b_f2.reshape(1, NT))
    return out
